# baseline (speedup 1.0000x reference)
import jax
import jax.numpy as jnp
from jax.experimental import pallas as pl
from jax.experimental.pallas import tpu as pltpu

N = 256
D = 256
E = 4096
EB = 1024
NBLK = E // EB
EPS = 1e-06
F32 = jnp.float32


def _fused_body(conn_ref, rel_ref, conn_all_ref, phr_ref,
                wrel_ref, wsub_ref, wobj_ref, wphr_ref, brel_ref, bsub_ref,
                bobj_ref, bphr_ref, t_ref, upd_ref, out_ref, tsub_s, tobj_s,
                upd_s, erow_s, ecol_s):
    i = pl.program_id(0)

    @pl.when(i == 0)
    def _tables():
        phr = phr_ref[...]
        tsub_s[:, :D] = jnp.dot(phr, wrel_ref[:D], preferred_element_type=F32)
        tsub_s[:, D:] = jnp.dot(phr, wsub_ref[:D], preferred_element_type=F32)
        tobj_s[:, :D] = jnp.dot(phr, wrel_ref[D:2 * D],
                                preferred_element_type=F32)
        tobj_s[:, D:] = jnp.dot(phr, wobj_ref[:D], preferred_element_type=F32)
        t = t_ref[0, 0]
        sub_all = conn_all_ref[0, :]
        obj_all = conn_all_ref[1, :]
        iota_e = jax.lax.broadcasted_iota(jnp.int32, (E, N), 0)
        iota_o = jax.lax.broadcasted_iota(jnp.int32, (E, N), 1)
        subc = sub_all[:, None]
        objc = obj_all[:, None]
        rowval = jnp.where(subc == t, iota_e[:, 0:1], -1)
        colval = jnp.where(objc == t, iota_e[:, 0:1], -1)
        erow_s[0, :] = jnp.max(jnp.where(objc == iota_o, rowval, -1), axis=0)
        ecol_s[0, :] = jnp.max(jnp.where(subc == iota_o, colval, -1), axis=0)

    sub = conn_ref[0, :]
    obj = conn_ref[1, :]
    iota_n = jax.lax.broadcasted_iota(jnp.int32, (EB, N), 1)
    oh_sub = (sub[:, None] == iota_n).astype(F32)
    oh_obj = (obj[:, None] == iota_n).astype(F32)
    upd = (jnp.dot(oh_sub, tsub_s[:, :D], preferred_element_type=F32)
           + jnp.dot(oh_obj, tobj_s[:, :D], preferred_element_type=F32)
           + brel_ref[...][None, :]
           + jnp.dot(rel_ref[...], wrel_ref[2 * D:],
                     preferred_element_type=F32))
    upd_ref[...] = upd
    upd_s[pl.ds(i * EB, EB), :] = upd

    @pl.when(i == NBLK - 1)
    def _context():
        t = t_ref[0, 0]
        iota_e = jax.lax.broadcasted_iota(jnp.int32, (E, N), 0)
        e_row = erow_s[0, :]
        e_col = ecol_s[0, :]
        sel_row = (iota_e == e_row[None, :]).astype(F32)
        sel_col = (iota_e == e_col[None, :]).astype(F32)
        r_row = jax.lax.dot_general(sel_row, upd_s[...],
                                    (((0,), (0,)), ((), ())),
                                    preferred_element_type=F32)
        r_col = jax.lax.dot_general(sel_col, upd_s[...],
                                    (((0,), (0,)), ((), ())),
                                    preferred_element_type=F32)
        oh_t = (jax.lax.broadcasted_iota(jnp.int32, (1, N), 1) == t
                ).astype(F32)
        ps_t = jnp.dot(oh_t, tsub_s[:, D:], preferred_element_type=F32)
        po_t = jnp.dot(oh_t, tobj_s[:, D:], preferred_element_type=F32)
        bsub = bsub_ref[...][None, :]
        bobj = bobj_ref[...][None, :]
        scale = 1.0 / (D ** 0.5)
        ts_row = ps_t + bsub + jnp.dot(r_row, wsub_ref[D:],
                                       preferred_element_type=F32)
        to_row = tobj_s[:, D:] + bobj + jnp.dot(r_row, wobj_ref[D:],
                                                preferred_element_type=F32)
        a_row = jnp.sum(ts_row * to_row, axis=1) * scale
        ts_col = tsub_s[:, D:] + bsub + jnp.dot(r_col, wsub_ref[D:],
                                                preferred_element_type=F32)
        to_col = po_t + bobj + jnp.dot(r_col, wobj_ref[D:],
                                       preferred_element_type=F32)
        a_col = jnp.sum(ts_col * to_col, axis=1) * scale
        mask_row = (e_row >= 0).astype(F32)
        mask_col = (e_col >= 0).astype(F32)

        def msm(vec, mask):
            mv = vec * mask
            ex = jnp.exp(mv - jnp.max(mv)) * mask
            return ex / (jnp.sum(ex) + EPS)

        w_row = msm(a_row, mask_row)
        w_col = msm(a_col, mask_col)
        ctx1 = jnp.dot((w_row + w_col)[None, :], phr_ref[...],
                       preferred_element_type=F32)
        ctx2 = (jnp.dot(w_row[None, :], r_row, preferred_element_type=F32)
                + jnp.dot(w_col[None, :], r_col, preferred_element_type=F32))
        delta = (jnp.dot(ctx1, wphr_ref[:D], preferred_element_type=F32)
                 + jnp.dot(ctx2, wphr_ref[D:], preferred_element_type=F32)
                 + bphr_ref[...][None, :])
        row_is_t = jax.lax.broadcasted_iota(jnp.int32, (N, 1), 0) == t
        out_ref[...] = phr_ref[...] + jnp.where(row_is_t, delta, 0.0)


@jax.jit
def _run(phrase_feat, rel_feat, rel_conn_mat, target_id, W_rel, b_rel, W_sub,
         b_sub, W_obj, b_obj, W_phr, b_phr):
    conn = rel_conn_mat.astype(jnp.int32)
    t = jnp.asarray(target_id, jnp.int32).reshape(1, 1)

    full = lambda shape: pl.BlockSpec(shape, lambda i: tuple(0 for _ in shape))
    upd, out1 = pl.pallas_call(
        _fused_body,
        grid=(NBLK,),
        in_specs=[
            pl.BlockSpec((2, EB), lambda i: (0, i)),
            pl.BlockSpec((EB, D), lambda i: (i, 0)),
            full((2, E)),
            full((N, D)),
            full((3 * D, D)),
            full((2 * D, D)),
            full((2 * D, D)),
            full((2 * D, D)),
            full((D,)),
            full((D,)),
            full((D,)),
            full((D,)),
            full((1, 1)),
        ],
        out_specs=(pl.BlockSpec((EB, D), lambda i: (i, 0)), full((N, D))),
        out_shape=(jax.ShapeDtypeStruct((E, D), F32),
                   jax.ShapeDtypeStruct((N, D), F32)),
        scratch_shapes=[
            pltpu.VMEM((N, 2 * D), F32),
            pltpu.VMEM((N, 2 * D), F32),
            pltpu.VMEM((E, D), F32),
            pltpu.VMEM((1, N), jnp.int32),
            pltpu.VMEM((1, N), jnp.int32),
        ],
    )(conn, rel_feat, conn, phrase_feat, W_rel, W_sub, W_obj, W_phr,
      b_rel, b_sub, b_obj, b_phr, t)
    return out1, upd


def kernel(phrase_feat, rel_feat, rel_conn_mat, target_id, W_rel, b_rel,
           W_sub, b_sub, W_obj, b_obj, W_phr, b_phr):
    return _run(phrase_feat, rel_feat, rel_conn_mat, target_id, W_rel, b_rel,
                W_sub, b_sub, W_obj, b_obj, W_phr, b_phr)


# speedup vs baseline: 1.1186x; 1.1186x over previous
import jax
import jax.numpy as jnp
from jax.experimental import pallas as pl
from jax.experimental.pallas import tpu as pltpu

N = 256
D = 256
E = 4096
EB = 4096
NBLK = E // EB
EPS = 1e-06
F32 = jnp.float32


def _fused_body(conn_ref, rel_ref, conn_all_ref, phr_ref,
                wrel_ref, wsub_ref, wobj_ref, wphr_ref, brel_ref, bsub_ref,
                bobj_ref, bphr_ref, t_ref, upd_ref, out_ref, tsub_s, tobj_s,
                upd_s, erow_s, ecol_s):
    i = pl.program_id(0)

    @pl.when(i == 0)
    def _tables():
        phr = phr_ref[...]
        tsub_s[:, :D] = jnp.dot(phr, wrel_ref[:D], preferred_element_type=F32)
        tsub_s[:, D:] = jnp.dot(phr, wsub_ref[:D], preferred_element_type=F32)
        tobj_s[:, :D] = jnp.dot(phr, wrel_ref[D:2 * D],
                                preferred_element_type=F32)
        tobj_s[:, D:] = jnp.dot(phr, wobj_ref[:D], preferred_element_type=F32)
        t = t_ref[0, 0]
        sub_all = conn_all_ref[0, :]
        obj_all = conn_all_ref[1, :]
        iota_e = jax.lax.broadcasted_iota(jnp.int32, (E, N), 0)
        iota_o = jax.lax.broadcasted_iota(jnp.int32, (E, N), 1)
        subc = sub_all[:, None]
        objc = obj_all[:, None]
        rowval = jnp.where(subc == t, iota_e[:, 0:1], -1)
        colval = jnp.where(objc == t, iota_e[:, 0:1], -1)
        erow_s[0, :] = jnp.max(jnp.where(objc == iota_o, rowval, -1), axis=0)
        ecol_s[0, :] = jnp.max(jnp.where(subc == iota_o, colval, -1), axis=0)

    sub = conn_ref[0, :]
    obj = conn_ref[1, :]
    iota_n = jax.lax.broadcasted_iota(jnp.int32, (EB, N), 1)
    oh_sub = (sub[:, None] == iota_n).astype(F32)
    oh_obj = (obj[:, None] == iota_n).astype(F32)
    upd = (jnp.dot(oh_sub, tsub_s[:, :D], preferred_element_type=F32)
           + jnp.dot(oh_obj, tobj_s[:, :D], preferred_element_type=F32)
           + brel_ref[...][None, :]
           + jnp.dot(rel_ref[...], wrel_ref[2 * D:],
                     preferred_element_type=F32))
    upd_ref[...] = upd
    upd_s[pl.ds(i * EB, EB), :] = upd

    @pl.when(i == NBLK - 1)
    def _context():
        t = t_ref[0, 0]
        iota_e = jax.lax.broadcasted_iota(jnp.int32, (E, N), 0)
        e_row = erow_s[0, :]
        e_col = ecol_s[0, :]
        sel_row = (iota_e == e_row[None, :]).astype(F32)
        sel_col = (iota_e == e_col[None, :]).astype(F32)
        r_row = jax.lax.dot_general(sel_row, upd_s[...],
                                    (((0,), (0,)), ((), ())),
                                    preferred_element_type=F32)
        r_col = jax.lax.dot_general(sel_col, upd_s[...],
                                    (((0,), (0,)), ((), ())),
                                    preferred_element_type=F32)
        oh_t = (jax.lax.broadcasted_iota(jnp.int32, (1, N), 1) == t
                ).astype(F32)
        ps_t = jnp.dot(oh_t, tsub_s[:, D:], preferred_element_type=F32)
        po_t = jnp.dot(oh_t, tobj_s[:, D:], preferred_element_type=F32)
        bsub = bsub_ref[...][None, :]
        bobj = bobj_ref[...][None, :]
        scale = 1.0 / (D ** 0.5)
        ts_row = ps_t + bsub + jnp.dot(r_row, wsub_ref[D:],
                                       preferred_element_type=F32)
        to_row = tobj_s[:, D:] + bobj + jnp.dot(r_row, wobj_ref[D:],
                                                preferred_element_type=F32)
        a_row = jnp.sum(ts_row * to_row, axis=1) * scale
        ts_col = tsub_s[:, D:] + bsub + jnp.dot(r_col, wsub_ref[D:],
                                                preferred_element_type=F32)
        to_col = po_t + bobj + jnp.dot(r_col, wobj_ref[D:],
                                       preferred_element_type=F32)
        a_col = jnp.sum(ts_col * to_col, axis=1) * scale
        mask_row = (e_row >= 0).astype(F32)
        mask_col = (e_col >= 0).astype(F32)

        def msm(vec, mask):
            mv = vec * mask
            ex = jnp.exp(mv - jnp.max(mv)) * mask
            return ex / (jnp.sum(ex) + EPS)

        w_row = msm(a_row, mask_row)
        w_col = msm(a_col, mask_col)
        ctx1 = jnp.dot((w_row + w_col)[None, :], phr_ref[...],
                       preferred_element_type=F32)
        ctx2 = (jnp.dot(w_row[None, :], r_row, preferred_element_type=F32)
                + jnp.dot(w_col[None, :], r_col, preferred_element_type=F32))
        delta = (jnp.dot(ctx1, wphr_ref[:D], preferred_element_type=F32)
                 + jnp.dot(ctx2, wphr_ref[D:], preferred_element_type=F32)
                 + bphr_ref[...][None, :])
        row_is_t = jax.lax.broadcasted_iota(jnp.int32, (N, 1), 0) == t
        out_ref[...] = phr_ref[...] + jnp.where(row_is_t, delta, 0.0)


@jax.jit
def _run(phrase_feat, rel_feat, rel_conn_mat, target_id, W_rel, b_rel, W_sub,
         b_sub, W_obj, b_obj, W_phr, b_phr):
    conn = rel_conn_mat.astype(jnp.int32)
    t = jnp.asarray(target_id, jnp.int32).reshape(1, 1)

    full = lambda shape: pl.BlockSpec(shape, lambda i: tuple(0 for _ in shape))
    upd, out1 = pl.pallas_call(
        _fused_body,
        grid=(NBLK,),
        in_specs=[
            pl.BlockSpec((2, EB), lambda i: (0, i)),
            pl.BlockSpec((EB, D), lambda i: (i, 0)),
            full((2, E)),
            full((N, D)),
            full((3 * D, D)),
            full((2 * D, D)),
            full((2 * D, D)),
            full((2 * D, D)),
            full((D,)),
            full((D,)),
            full((D,)),
            full((D,)),
            full((1, 1)),
        ],
        out_specs=(pl.BlockSpec((EB, D), lambda i: (i, 0)), full((N, D))),
        out_shape=(jax.ShapeDtypeStruct((E, D), F32),
                   jax.ShapeDtypeStruct((N, D), F32)),
        scratch_shapes=[
            pltpu.VMEM((N, 2 * D), F32),
            pltpu.VMEM((N, 2 * D), F32),
            pltpu.VMEM((E, D), F32),
            pltpu.VMEM((1, N), jnp.int32),
            pltpu.VMEM((1, N), jnp.int32),
        ],
    )(conn, rel_feat, conn, phrase_feat, W_rel, W_sub, W_obj, W_phr,
      b_rel, b_sub, b_obj, b_phr, t)
    return out1, upd


def kernel(phrase_feat, rel_feat, rel_conn_mat, target_id, W_rel, b_rel,
           W_sub, b_sub, W_obj, b_obj, W_phr, b_phr):
    return _run(phrase_feat, rel_feat, rel_conn_mat, target_id, W_rel, b_rel,
                W_sub, b_sub, W_obj, b_obj, W_phr, b_phr)


# speedup vs baseline: 1.1498x; 1.0279x over previous
import jax
import jax.numpy as jnp
from jax.experimental import pallas as pl

N = 256
D = 256
E = 4096
EPS = 1e-06
F32 = jnp.float32


def _fused_body(conn_ref, rel_ref, phr_ref, wrel_ref, wsub_ref, wobj_ref,
                wphr_ref, brel_ref, bsub_ref, bobj_ref, bphr_ref, t_ref,
                upd_ref, out_ref):
    t = t_ref[0, 0]
    phr = phr_ref[...]
    pa = jnp.dot(phr, wrel_ref[:D], preferred_element_type=F32)
    ps = jnp.dot(phr, wsub_ref[:D], preferred_element_type=F32)
    pb = jnp.dot(phr, wrel_ref[D:2 * D], preferred_element_type=F32)
    po = jnp.dot(phr, wobj_ref[:D], preferred_element_type=F32)

    sub = conn_ref[0, :]
    obj = conn_ref[1, :]
    subc = sub[:, None]
    objc = obj[:, None]
    iota_eo = jax.lax.broadcasted_iota(jnp.int32, (E, N), 1)
    iota_ee = jax.lax.broadcasted_iota(jnp.int32, (E, N), 0)

    rowval = jnp.where(subc == t, iota_ee[:, 0:1], -1)
    colval = jnp.where(objc == t, iota_ee[:, 0:1], -1)
    oh_sub_m = subc == iota_eo
    oh_obj_m = objc == iota_eo
    e_row = jnp.max(jnp.where(oh_obj_m, rowval, -1), axis=0)
    e_col = jnp.max(jnp.where(oh_sub_m, colval, -1), axis=0)

    oh_sub = oh_sub_m.astype(F32)
    oh_obj = oh_obj_m.astype(F32)
    upd = (jnp.dot(oh_sub, pa, preferred_element_type=F32)
           + jnp.dot(oh_obj, pb, preferred_element_type=F32)
           + brel_ref[...][None, :]
           + jnp.dot(rel_ref[...], wrel_ref[2 * D:],
                     preferred_element_type=F32))
    upd_ref[...] = upd

    sel_row = (iota_ee == e_row[None, :]).astype(F32)
    sel_col = (iota_ee == e_col[None, :]).astype(F32)
    r_row = jax.lax.dot_general(sel_row, upd, (((0,), (0,)), ((), ())),
                                preferred_element_type=F32)
    r_col = jax.lax.dot_general(sel_col, upd, (((0,), (0,)), ((), ())),
                                preferred_element_type=F32)

    oh_t = (jax.lax.broadcasted_iota(jnp.int32, (1, N), 1) == t).astype(F32)
    ps_t = jnp.dot(oh_t, ps, preferred_element_type=F32)
    po_t = jnp.dot(oh_t, po, preferred_element_type=F32)
    bsub = bsub_ref[...][None, :]
    bobj = bobj_ref[...][None, :]
    scale = 1.0 / (D ** 0.5)
    ts_row = ps_t + bsub + jnp.dot(r_row, wsub_ref[D:],
                                   preferred_element_type=F32)
    to_row = po + bobj + jnp.dot(r_row, wobj_ref[D:],
                                 preferred_element_type=F32)
    a_row = jnp.sum(ts_row * to_row, axis=1) * scale
    ts_col = ps + bsub + jnp.dot(r_col, wsub_ref[D:],
                                 preferred_element_type=F32)
    to_col = po_t + bobj + jnp.dot(r_col, wobj_ref[D:],
                                   preferred_element_type=F32)
    a_col = jnp.sum(ts_col * to_col, axis=1) * scale
    mask_row = (e_row >= 0).astype(F32)
    mask_col = (e_col >= 0).astype(F32)

    def msm(vec, mask):
        mv = vec * mask
        ex = jnp.exp(mv - jnp.max(mv)) * mask
        return ex / (jnp.sum(ex) + EPS)

    w_row = msm(a_row, mask_row)
    w_col = msm(a_col, mask_col)
    ctx1 = jnp.dot((w_row + w_col)[None, :], phr, preferred_element_type=F32)
    ctx2 = (jnp.dot(w_row[None, :], r_row, preferred_element_type=F32)
            + jnp.dot(w_col[None, :], r_col, preferred_element_type=F32))
    delta = (jnp.dot(ctx1, wphr_ref[:D], preferred_element_type=F32)
             + jnp.dot(ctx2, wphr_ref[D:], preferred_element_type=F32)
             + bphr_ref[...][None, :])
    row_is_t = jax.lax.broadcasted_iota(jnp.int32, (N, 1), 0) == t
    out_ref[...] = phr + jnp.where(row_is_t, delta, 0.0)


@jax.jit
def _run(phrase_feat, rel_feat, rel_conn_mat, target_id, W_rel, b_rel, W_sub,
         b_sub, W_obj, b_obj, W_phr, b_phr):
    conn = rel_conn_mat.astype(jnp.int32)
    t = jnp.asarray(target_id, jnp.int32).reshape(1, 1)
    upd, out1 = pl.pallas_call(
        _fused_body,
        out_shape=(jax.ShapeDtypeStruct((E, D), F32),
                   jax.ShapeDtypeStruct((N, D), F32)),
    )(conn, rel_feat, phrase_feat, W_rel, W_sub, W_obj, W_phr,
      b_rel, b_sub, b_obj, b_phr, t)
    return out1, upd


def kernel(phrase_feat, rel_feat, rel_conn_mat, target_id, W_rel, b_rel,
           W_sub, b_sub, W_obj, b_obj, W_phr, b_phr):
    return _run(phrase_feat, rel_feat, rel_conn_mat, target_id, W_rel, b_rel,
                W_sub, b_sub, W_obj, b_obj, W_phr, b_phr)
